# baseline (speedup 1.0000x reference)
import functools

import jax
import jax.numpy as jnp
from jax import lax
from jax.experimental import pallas as pl
from jax.experimental.pallas import tpu as pltpu
from jax.experimental.pallas import tpu_sc as plsc

DIM = 64
L01 = 10
L23 = 12
LANES = 16
NV = DIM // LANES
NC = 2
NS = 16
NW = NC * NS


@functools.lru_cache(maxsize=None)
def _make_sc_kernel(batch: int, lf: int):
    spt = batch // NW
    c = 8
    nch = spt // c

    mesh = plsc.VectorSubcoreMesh(core_axis_name="c", subcore_axis_name="s")

    @functools.partial(
        pl.kernel,
        out_type=jax.ShapeDtypeStruct((batch, 2 * DIM), jnp.float32),
        mesh=mesh,
        compiler_params=pltpu.CompilerParams(use_tc_tiling_on_sc=False),
        scratch_types=[
            pltpu.VMEM((2 * nch * c * lf,), jnp.int32),
            pltpu.VMEM((4, c * lf, DIM), jnp.float32),
            pltpu.VMEM((4, c * lf, DIM), jnp.float32),
            pltpu.VMEM((2, LANES), jnp.float32),
            pltpu.VMEM((2, c, 2 * DIM), jnp.float32),
            pltpu.SemaphoreType.DMA,
            pltpu.SemaphoreType.DMA,
            pltpu.SemaphoreType.DMA,
            pltpu.SemaphoreType.DMA,
            pltpu.SemaphoreType.DMA,
            pltpu.SemaphoreType.DMA,
        ],
    )
    def k(ih, th, wh, outh,
          iv, r0v, r1v, wv, ov, sem0, sem1, sem2, sem3, semo0, semo1):
        cid = lax.axis_index("c")
        sid = lax.axis_index("s")
        wid = sid * NC + cid
        base = wid * spt
        seg = nch * c * lf

        pltpu.sync_copy(ih.at[wid], iv)
        pltpu.sync_copy(wh, wv)

        feats = ((0, r0v, 0), (seg, r1v, 1))
        sems = (sem0, sem1, sem2, sem3)
        semos = (semo0, semo1)

        wvals = []
        for (_, _, f) in feats:
            wrow = wv[f]
            wvals.append([wrow[j] for j in range(lf)])

        def gathers(g, slot):
            for (off, rv, _) in feats:
                pltpu.make_async_copy(
                    th.at[iv.at[pl.ds(off + g * c * lf, c * lf)]],
                    rv.at[slot], sems[slot]).start()

        def drains(slot):
            for (off, rv, _) in feats:
                pltpu.make_async_copy(
                    th.at[iv.at[pl.ds(off, c * lf)]],
                    rv.at[slot], sems[slot]).wait()

        def out_copy(g, oslot):
            return pltpu.make_async_copy(
                ov.at[oslot], outh.at[pl.ds(base + g * c, c)], semos[oslot])

        def compute(g, slot, oslot):
            def one(s):
                for (_, rv, f) in feats:
                    acc = [rv[slot, s * lf, pl.ds(v * LANES, LANES)]
                           * wvals[f][0] for v in range(NV)]
                    for j in range(1, lf):
                        for v in range(NV):
                            acc[v] = acc[v] + (rv[slot, s * lf + j,
                                                  pl.ds(v * LANES, LANES)]
                                               * wvals[f][j])
                    for v in range(NV):
                        ov[oslot, s, pl.ds(f * DIM + v * LANES, LANES)] = acc[v]

            def spair(q, carry):
                one(2 * q)
                one(2 * q + 1)
                return carry
            lax.fori_loop(0, c // 2, spair, 0)

        gathers(0, 0)
        gathers(1, 1)
        gathers(2, 2)

        def quad(p, carry):
            for slot in (0, 1, 2, 3):
                g = 4 * p + slot
                oslot = slot % 2

                @pl.when(g + 3 < nch)
                def _():
                    gathers(g + 3, (slot + 3) % 4)

                drains(slot)

                @pl.when(g >= 2)
                def _():
                    out_copy(g, oslot).wait()

                compute(g, slot, oslot)
                out_copy(g, oslot).start()
            return carry
        lax.fori_loop(0, nch // 4, quad, 0)
        out_copy(nch - 2, 0).wait()
        out_copy(nch - 1, 1).wait()

    return k


def kernel(idx_f0, idx_f1, idx_f2, idx_f3, table_0, table_1,
           pos_w_0, pos_w_1, pos_w_2, pos_w_3):
    batch = idx_f0.shape[0]
    spt = batch // NW
    ia = jnp.concatenate([idx_f0.reshape(NW, spt * L01),
                          idx_f1.reshape(NW, spt * L01)], axis=1)
    ib = jnp.concatenate([idx_f2.reshape(NW, spt * L23),
                          idx_f3.reshape(NW, spt * L23)], axis=1)
    wa = jnp.zeros((2, LANES), jnp.float32)
    wa = wa.at[0, :L01].set(pos_w_0).at[1, :L01].set(pos_w_1)
    wb = jnp.zeros((2, LANES), jnp.float32)
    wb = wb.at[0, :L23].set(pos_w_2).at[1, :L23].set(pos_w_3)
    pred_b = _make_sc_kernel(batch, L23)(ib, table_1, wb)
    pred_a = _make_sc_kernel(batch, L01)(ia, table_0, wa)
    loss = (jnp.sum(pred_b) + jnp.sum(pred_a)) / (batch * 4 * DIM)
    pred = jnp.concatenate([pred_a, pred_b], axis=1)
    return (loss, pred)


# speedup vs baseline: 1.0070x; 1.0070x over previous
import functools

import jax
import jax.numpy as jnp
from jax import lax
from jax.experimental import pallas as pl
from jax.experimental.pallas import tpu as pltpu
from jax.experimental.pallas import tpu_sc as plsc

DIM = 64
L01 = 10
L23 = 12
LANES = 16
NV = DIM // LANES
NC = 2
NS = 16
NW = NC * NS


@functools.lru_cache(maxsize=None)
def _make_sc_kernel(batch: int, lf: int):
    spt = batch // NW
    c = 8
    nch = spt // c

    mesh = plsc.VectorSubcoreMesh(core_axis_name="c", subcore_axis_name="s")

    @functools.partial(
        pl.kernel,
        out_type=jax.ShapeDtypeStruct((batch, 2 * DIM), jnp.float32),
        mesh=mesh,
        compiler_params=pltpu.CompilerParams(use_tc_tiling_on_sc=False),
        scratch_types=[
            pltpu.VMEM((2 * nch * c * lf,), jnp.int32),
            pltpu.VMEM((4, c * lf, DIM), jnp.float32),
            pltpu.VMEM((4, c * lf, DIM), jnp.float32),
            pltpu.VMEM((2, LANES), jnp.float32),
            pltpu.VMEM((2, c, 2 * DIM), jnp.float32),
            pltpu.SemaphoreType.DMA,
            pltpu.SemaphoreType.DMA,
            pltpu.SemaphoreType.DMA,
            pltpu.SemaphoreType.DMA,
            pltpu.SemaphoreType.DMA,
            pltpu.SemaphoreType.DMA,
        ],
    )
    def k(ih, th, wh, outh,
          iv, r0v, r1v, wv, ov, sem0, sem1, sem2, sem3, semo0, semo1):
        cid = lax.axis_index("c")
        sid = lax.axis_index("s")
        wid = sid * NC + cid
        base = wid * spt
        seg = nch * c * lf

        pltpu.sync_copy(ih.at[wid], iv)
        pltpu.sync_copy(wh, wv)

        feats = ((0, r0v, 0), (seg, r1v, 1))
        sems = (sem0, sem1, sem2, sem3)
        semos = (semo0, semo1)

        wvals = []
        for (_, _, f) in feats:
            wrow = wv[f]
            wvals.append([wrow[j] for j in range(lf)])

        def gathers(g, slot):
            for (off, rv, _) in feats:
                pltpu.make_async_copy(
                    th.at[iv.at[pl.ds(off + g * c * lf, c * lf)]],
                    rv.at[slot], sems[slot]).start()

        def drains(slot):
            for (off, rv, _) in feats:
                pltpu.make_async_copy(
                    th.at[iv.at[pl.ds(off, c * lf)]],
                    rv.at[slot], sems[slot]).wait()

        def out_copy(g, oslot):
            return pltpu.make_async_copy(
                ov.at[oslot], outh.at[pl.ds(base + g * c, c)], semos[oslot])

        def compute(g, slot, oslot):
            def sample(s, carry):
                for (_, rv, f) in feats:
                    acc = [rv[slot, s * lf, pl.ds(v * LANES, LANES)]
                           * wvals[f][0] for v in range(NV)]
                    for j in range(1, lf):
                        for v in range(NV):
                            acc[v] = acc[v] + (rv[slot, s * lf + j,
                                                  pl.ds(v * LANES, LANES)]
                                               * wvals[f][j])
                    for v in range(NV):
                        ov[oslot, s, pl.ds(f * DIM + v * LANES, LANES)] = acc[v]
                return carry
            lax.fori_loop(0, c, sample, 0)

        gathers(0, 0)
        gathers(1, 1)
        gathers(2, 2)

        def quad(p, carry):
            for slot in (0, 1, 2, 3):
                g = 4 * p + slot
                oslot = slot % 2

                @pl.when(g + 3 < nch)
                def _():
                    gathers(g + 3, (slot + 3) % 4)

                drains(slot)

                @pl.when(g >= 2)
                def _():
                    out_copy(g, oslot).wait()

                compute(g, slot, oslot)
                out_copy(g, oslot).start()
            return carry
        lax.fori_loop(0, nch // 4, quad, 0)
        out_copy(nch - 2, 0).wait()
        out_copy(nch - 1, 1).wait()

    return k


def kernel(idx_f0, idx_f1, idx_f2, idx_f3, table_0, table_1,
           pos_w_0, pos_w_1, pos_w_2, pos_w_3):
    batch = idx_f0.shape[0]
    spt = batch // NW
    ia = jnp.concatenate([idx_f0.reshape(NW, spt * L01),
                          idx_f1.reshape(NW, spt * L01)], axis=1)
    ib = jnp.concatenate([idx_f2.reshape(NW, spt * L23),
                          idx_f3.reshape(NW, spt * L23)], axis=1)
    wa = jnp.zeros((2, LANES), jnp.float32)
    wa = wa.at[0, :L01].set(pos_w_0).at[1, :L01].set(pos_w_1)
    wb = jnp.zeros((2, LANES), jnp.float32)
    wb = wb.at[0, :L23].set(pos_w_2).at[1, :L23].set(pos_w_3)
    pred_b = _make_sc_kernel(batch, L23)(ib, table_1, wb)
    pred_a = _make_sc_kernel(batch, L01)(ia, table_0, wa)
    loss = (jnp.sum(pred_b) + jnp.sum(pred_a)) / (batch * 4 * DIM)
    pred = jnp.concatenate([pred_a, pred_b], axis=1)
    return (loss, pred)
